# Initial kernel scaffold; baseline (speedup 1.0000x reference)
import jax
import jax.numpy as jnp
from jax.experimental import pallas as pl


def kernel(indices, embedding):
    raise NotImplementedError("write your pallas kernel here")


# speedup vs baseline: 1.1027x; 1.1027x over previous
import functools

import jax
import jax.numpy as jnp
from jax import lax
from jax.experimental import pallas as pl
from jax.experimental.pallas import tpu as pltpu
from jax.experimental.pallas import tpu_sc as plsc

_NUM_CORES = 2
_NUM_SUBCORES = 16
_NUM_WORKERS = _NUM_CORES * _NUM_SUBCORES

_CHUNK = 1600


def _gather_kernel(n_total, d, idx_hbm, table_hbm, out_hbm, idx_v, rows_v, sem):
    per_w = n_total // _NUM_WORKERS
    n_chunks = per_w // _CHUNK
    wid = lax.axis_index("s") * _NUM_CORES + lax.axis_index("c")
    base = wid * per_w

    def body(i, carry):
        off = base + i * _CHUNK
        pltpu.sync_copy(idx_hbm.at[pl.ds(off, _CHUNK)], idx_v)
        pltpu.async_copy(table_hbm.at[idx_v], rows_v, sem).wait()
        pltpu.sync_copy(rows_v, out_hbm.at[pl.ds(off, _CHUNK)])
        return carry

    lax.fori_loop(0, n_chunks, body, 0)


def kernel(indices, embedding):
    b, h = indices.shape
    v, d = embedding.shape
    n = b * h
    assert n % (_NUM_WORKERS * _CHUNK) == 0

    flat_idx = indices.reshape(n).astype(jnp.int32)

    mesh = plsc.VectorSubcoreMesh(core_axis_name="c", subcore_axis_name="s")
    run = functools.partial(
        pl.kernel,
        mesh=mesh,
        out_type=jax.ShapeDtypeStruct((n, d), jnp.float32),
        scratch_types=[
            pltpu.VMEM((_CHUNK,), jnp.int32),
            pltpu.VMEM((_CHUNK, d), jnp.float32),
            pltpu.SemaphoreType.DMA,
        ],
        compiler_params=pltpu.CompilerParams(use_tc_tiling_on_sc=False),
    )(functools.partial(_gather_kernel, n, d))

    out = run(flat_idx, embedding)
    return out.reshape(b, h, d)


# speedup vs baseline: 1.1137x; 1.0099x over previous
import functools

import jax
import jax.numpy as jnp
from jax import lax
from jax.experimental import pallas as pl
from jax.experimental.pallas import tpu as pltpu
from jax.experimental.pallas import tpu_sc as plsc

_NUM_CORES = 2
_NUM_SUBCORES = 16
_NUM_WORKERS = _NUM_CORES * _NUM_SUBCORES

_CHUNK = 1600


def _gather_kernel(n_total, d, idx_hbm, table_hbm, out_hbm,
                   idx_v, rows0, rows1, sem_g0, sem_g1, sem_o0, sem_o1):
    per_w = n_total // _NUM_WORKERS
    n_chunks = per_w // _CHUNK
    wid = lax.axis_index("s") * _NUM_CORES + lax.axis_index("c")
    base = wid * per_w

    rows = (rows0, rows1)
    sem_g = (sem_g0, sem_g1)
    sem_o = (sem_o0, sem_o1)

    pltpu.sync_copy(idx_hbm.at[pl.ds(base, per_w)], idx_v)

    def gather(i, b):
        return pltpu.make_async_copy(
            table_hbm.at[idx_v.at[pl.ds(i * _CHUNK, _CHUNK)]], rows[b], sem_g[b])

    def writeback(i, b):
        return pltpu.make_async_copy(
            rows[b], out_hbm.at[pl.ds(base + i * _CHUNK, _CHUNK)], sem_o[b])

    gather(0, 0).start()
    gather(1, 1).start()
    for i in range(n_chunks):
        b = i % 2
        gather(i, b).wait()
        writeback(i, b).start()
        if i + 2 < n_chunks:
            writeback(i, b).wait()
            gather(i + 2, b).start()
        elif i >= n_chunks - 2:
            writeback(i, b).wait()


def kernel(indices, embedding):
    b, h = indices.shape
    v, d = embedding.shape
    n = b * h
    assert n % (_NUM_WORKERS * _CHUNK) == 0

    flat_idx = indices.reshape(n).astype(jnp.int32)
    per_w = n // _NUM_WORKERS

    mesh = plsc.VectorSubcoreMesh(core_axis_name="c", subcore_axis_name="s")
    run = functools.partial(
        pl.kernel,
        mesh=mesh,
        out_type=jax.ShapeDtypeStruct((n, d), jnp.float32),
        scratch_types=[
            pltpu.VMEM((per_w,), jnp.int32),
            pltpu.VMEM((_CHUNK, d), jnp.float32),
            pltpu.VMEM((_CHUNK, d), jnp.float32),
            pltpu.SemaphoreType.DMA,
            pltpu.SemaphoreType.DMA,
            pltpu.SemaphoreType.DMA,
            pltpu.SemaphoreType.DMA,
        ],
        compiler_params=pltpu.CompilerParams(use_tc_tiling_on_sc=False),
    )(functools.partial(_gather_kernel, n, d))

    out = run(flat_idx, embedding)
    return out.reshape(b, h, d)


# speedup vs baseline: 1.8052x; 1.6210x over previous
import functools

import jax
import jax.numpy as jnp
from jax import lax
from jax.experimental import pallas as pl
from jax.experimental.pallas import tpu as pltpu
from jax.experimental.pallas import tpu_sc as plsc

_NUM_CORES = 2
_NUM_SUBCORES = 16
_NUM_WORKERS = _NUM_CORES * _NUM_SUBCORES

_ROWS_PER_STEP = 32


def _gather_kernel(b, h, d, idx_hbm, table_hbm, out_hbm,
                   idx_v, rows0, rows1, sem_g0, sem_g1, sem_o0, sem_o1):
    rows_per_w = b // _NUM_WORKERS
    n_steps = rows_per_w // _ROWS_PER_STEP
    chunk = _ROWS_PER_STEP * h
    per_w = rows_per_w * h
    wid = lax.axis_index("s") * _NUM_CORES + lax.axis_index("c")
    base_row = wid * rows_per_w

    rows = (rows0, rows1)
    sem_g = (sem_g0, sem_g1)
    sem_o = (sem_o0, sem_o1)

    pltpu.sync_copy(idx_hbm.at[pl.ds(base_row * h, per_w)], idx_v)

    def gather(i, bf):
        return pltpu.make_async_copy(
            table_hbm.at[idx_v.at[pl.ds(i * chunk, chunk)]], rows[bf], sem_g[bf])

    def writeback_row(i, r, bf):
        return pltpu.make_async_copy(
            rows[bf].at[pl.ds(r * h, h)],
            out_hbm.at[base_row + i * _ROWS_PER_STEP + r], sem_o[bf])

    gather(0, 0).start()
    gather(1, 1).start()
    for i in range(n_steps):
        bf = i % 2
        gather(i, bf).wait()
        for r in range(_ROWS_PER_STEP):
            writeback_row(i, r, bf).start()
        for r in range(_ROWS_PER_STEP):
            writeback_row(i, r, bf).wait()
        if i + 2 < n_steps:
            gather(i + 2, bf).start()


def kernel(indices, embedding):
    b, h = indices.shape
    v, d = embedding.shape
    n = b * h
    assert b % (_NUM_WORKERS * _ROWS_PER_STEP) == 0

    flat_idx = indices.reshape(n).astype(jnp.int32)
    per_w = n // _NUM_WORKERS

    mesh = plsc.VectorSubcoreMesh(core_axis_name="c", subcore_axis_name="s")
    run = functools.partial(
        pl.kernel,
        mesh=mesh,
        out_type=jax.ShapeDtypeStruct((b, h, d), jnp.float32),
        scratch_types=[
            pltpu.VMEM((per_w,), jnp.int32),
            pltpu.VMEM((_ROWS_PER_STEP * h, d), jnp.float32),
            pltpu.VMEM((_ROWS_PER_STEP * h, d), jnp.float32),
            pltpu.SemaphoreType.DMA,
            pltpu.SemaphoreType.DMA,
            pltpu.SemaphoreType.DMA,
            pltpu.SemaphoreType.DMA,
        ],
        compiler_params=pltpu.CompilerParams(use_tc_tiling_on_sc=False),
    )(functools.partial(_gather_kernel, b, h, d))

    return run(flat_idx, embedding)


# speedup vs baseline: 2.1484x; 1.1901x over previous
import functools

import jax
import jax.numpy as jnp
from jax import lax
from jax.experimental import pallas as pl
from jax.experimental.pallas import tpu as pltpu
from jax.experimental.pallas import tpu_sc as plsc

_NUM_CORES = 2
_NUM_SUBCORES = 16
_NUM_WORKERS = _NUM_CORES * _NUM_SUBCORES


def _iota16():
    return jnp.arange(16, dtype=jnp.int32)


def _detile_kernel(v, tab_t, flat_out, in0, in1, out0, out1, in_tail,
                   out_tail, sem_r0, sem_r1, sem_w0, sem_w1, sem_t):
    nblk = v // 128
    tail = v - nblk * 128
    wid = lax.axis_index("s") * _NUM_CORES + lax.axis_index("c")
    cnt = (nblk - wid + _NUM_WORKERS - 1) // _NUM_WORKERS

    ins = (in0, in1)
    outs = (out0, out1)
    sem_r = (sem_r0, sem_r1)
    sem_w = (sem_w0, sem_w1)
    iota = _iota16()

    def read(k, bf):
        blk = (wid + k * _NUM_WORKERS) * 128
        return pltpu.make_async_copy(
            tab_t.at[:, pl.ds(blk, 128)], ins[bf], sem_r[bf])

    def write(k, bf):
        blk = wid + k * _NUM_WORKERS
        return pltpu.make_async_copy(
            outs[bf], flat_out.at[pl.ds(blk * 32, 32)], sem_w[bf])

    def transpose_block(src, dst, ncols):
        @plsc.parallel_loop(0, ncols, unroll=4)
        def _(q):
            qcol = jnp.full((16,), q, dtype=jnp.int32)
            v0 = plsc.load_gather(src, [iota, qcol])
            v1 = plsc.load_gather(src, [iota + 16, qcol])
            row = q // 4
            col = (q % 4) * 32
            dst[row, pl.ds(col, 16)] = v0
            dst[row, pl.ds(col + 16, 16)] = v1

    read(0, 0).start()
    read(1, 1).start()

    def body(i, c):
        for bf in range(2):
            k = 2 * i + bf

            @pl.when(k < cnt)
            def _():
                read(k, bf).wait()

                @pl.when(k >= 2)
                def _():
                    write(k - 2, bf).wait()

                transpose_block(ins[bf], outs[bf], 128)
                write(k, bf).start()

                @pl.when(k + 2 < cnt)
                def _():
                    read(k + 2, bf).start()
        return c

    lax.fori_loop(0, (cnt + 1) // 2, body, 0)

    write(0, 0).wait()
    write(0, 1).wait()

    if tail:
        @pl.when(wid == _NUM_WORKERS - 1)
        def _():
            pltpu.sync_copy(tab_t.at[:, pl.ds(nblk * 128, tail)], in_tail)
            transpose_block(in_tail, out_tail, tail)
            pltpu.make_async_copy(
                out_tail, flat_out.at[pl.ds(nblk * 32, tail // 4)],
                sem_t).start()
            pltpu.make_async_copy(
                out_tail, flat_out.at[pl.ds(nblk * 32, tail // 4)],
                sem_t).wait()


def _gather_kernel(b, h, idx_hbm, table_hbm, out5_hbm, idx_v, it0, it1,
                   rows0, rows1, tile0, tile1, sem_g0, sem_g1, sem_w0, sem_w1):
    rows_per_w = b // _NUM_WORKERS
    wid = lax.axis_index("s") * _NUM_CORES + lax.axis_index("c")

    idx_t = (it0, it1)
    rows = (rows0, rows1)
    tile = (tile0, tile1)
    sem_g = (sem_g0, sem_g1)
    sem_w = (sem_w0, sem_w1)
    iota = _iota16()
    d8_0 = iota // 8
    d8_1 = d8_0 + 2
    d8r = iota % 8

    pltpu.sync_copy(idx_hbm.at[pl.ds(wid * rows_per_w * h, rows_per_w * h)],
                    idx_v)

    def build_idx(t, bf):
        dst = idx_t[bf]
        for g in range(rows_per_w // 16):
            pos = (iota + g * 16) * h + t
            dst[pl.ds(g * 16, 16)] = plsc.load_gather(idx_v, [pos])

    def gather(bf):
        return pltpu.make_async_copy(
            table_hbm.at[idx_t[bf]], rows[bf], sem_g[bf])

    def write(t, bf):
        return pltpu.make_async_copy(
            tile[bf], out5_hbm.at[t, :, pl.ds(wid * 4, 4)], sem_w[bf])

    def transpose_rows(bf):
        src = rows[bf]
        dst = tile[bf]

        @plsc.parallel_loop(0, rows_per_w, unroll=4)
        def _(k):
            r0 = src[k, pl.ds(0, 16)]
            r1 = src[k, pl.ds(16, 16)]
            b128 = jnp.full((16,), k // 128, dtype=jnp.int32)
            b128r = jnp.full((16,), k % 128, dtype=jnp.int32)
            plsc.store_scatter(dst, [d8_0, b128, d8r, b128r], r0)
            plsc.store_scatter(dst, [d8_1, b128, d8r, b128r], r1)

    build_idx(0, 0)
    gather(0).start()
    build_idx(1, 1)
    gather(1).start()

    assert h % 2 == 0

    def body(i, c):
        for bf in range(2):
            t = 2 * i + bf
            gather(bf).wait()

            @pl.when(t >= 2)
            def _():
                write(t - 2, bf).wait()

            transpose_rows(bf)
            write(t, bf).start()

            @pl.when(t + 2 < h)
            def _():
                build_idx(t + 2, bf)
                gather(bf).start()
        return c

    lax.fori_loop(0, h // 2, body, 0)
    write(0, 0).wait()
    write(0, 1).wait()


def kernel(indices, embedding):
    b, h = indices.shape
    v, d = embedding.shape
    n = b * h
    assert d == 32 and b % (128 * _NUM_WORKERS) == 0 and v % 4 == 0

    flat_idx = indices.reshape(n).astype(jnp.int32)
    mesh = plsc.VectorSubcoreMesh(core_axis_name="c", subcore_axis_name="s")

    tail = v - (v // 128) * 128
    detile = functools.partial(
        pl.kernel,
        mesh=mesh,
        out_type=jax.ShapeDtypeStruct((v // 4, 128), jnp.float32),
        scratch_types=[
            pltpu.VMEM((32, 128), jnp.float32),
            pltpu.VMEM((32, 128), jnp.float32),
            pltpu.VMEM((32, 128), jnp.float32),
            pltpu.VMEM((32, 128), jnp.float32),
            pltpu.VMEM((32, max(tail, 1)), jnp.float32),
            pltpu.VMEM((max(tail // 4, 1), 128), jnp.float32),
            pltpu.SemaphoreType.DMA,
            pltpu.SemaphoreType.DMA,
            pltpu.SemaphoreType.DMA,
            pltpu.SemaphoreType.DMA,
            pltpu.SemaphoreType.DMA,
        ],
        compiler_params=pltpu.CompilerParams(use_tc_tiling_on_sc=True, needs_layout_passes=False),
    )(functools.partial(_detile_kernel, v))

    table_flat = detile(embedding.T).reshape(v, d)

    rows_per_w = b // _NUM_WORKERS
    gather_run = functools.partial(
        pl.kernel,
        mesh=mesh,
        out_type=jax.ShapeDtypeStruct((h, 4, b // 128, 8, 128), jnp.float32),
        scratch_types=[
            pltpu.VMEM((rows_per_w * h,), jnp.int32),
            pltpu.VMEM((rows_per_w,), jnp.int32),
            pltpu.VMEM((rows_per_w,), jnp.int32),
            pltpu.VMEM((rows_per_w, d), jnp.float32),
            pltpu.VMEM((rows_per_w, d), jnp.float32),
            pltpu.VMEM((4, 4, 8, 128), jnp.float32),
            pltpu.VMEM((4, 4, 8, 128), jnp.float32),
            pltpu.SemaphoreType.DMA,
            pltpu.SemaphoreType.DMA,
            pltpu.SemaphoreType.DMA,
            pltpu.SemaphoreType.DMA,
        ],
        compiler_params=pltpu.CompilerParams(use_tc_tiling_on_sc=False, needs_layout_passes=False),
    )(functools.partial(_gather_kernel, b, h))

    out5 = gather_run(flat_idx, table_flat)
    return out5.transpose(2, 4, 0, 1, 3).reshape(b, h, d)
